# Initial kernel scaffold; baseline (speedup 1.0000x reference)
import jax
import jax.numpy as jnp
from jax.experimental import pallas as pl


def kernel(X_drug, X_dis, X_gene, insts_rdr, insts_rdgdr, insts_dgd, insts_drd, insts_drrd, insts_gdrdg, W_rdr, a_rdr, W_rdgdr, a_rdgdr, W_dgd, a_dgd, W_drd, a_drd, W_drrd, a_drrd, W_gdrdg, a_gdrdg, mpW_drug, mpb_drug, mpq_drug, mpW_dis, mpb_dis, mpq_dis, mpW_gene, mpb_gene, mpq_gene):
    raise NotImplementedError("write your pallas kernel here")


# speedup vs baseline: 1.6350x; 1.6350x over previous
import functools

import jax
import jax.numpy as jnp
from jax import lax
from jax.experimental import pallas as pl
from jax.experimental.pallas import tpu as pltpu
from jax.experimental.pallas import tpu_sc as plsc

DIM = 128
SC_CHUNK = 128


def _build_sc_gather(total_pad, n_workers, table_rows):
  per_w = total_pad // n_workers
  iters = per_w // SC_CHUNK
  mesh = plsc.VectorSubcoreMesh(core_axis_name="c", subcore_axis_name="s")
  info = plsc.get_sparse_core_info()
  nc = info.num_cores

  @functools.partial(
      pl.kernel,
      mesh=mesh,
      out_type=jax.ShapeDtypeStruct((total_pad, DIM), jnp.float32),
      scratch_types=[
          pltpu.VMEM((SC_CHUNK,), jnp.int32),
          pltpu.VMEM((SC_CHUNK, DIM), jnp.float32),
          pltpu.SemaphoreType.DMA,
      ],
  )
  def sc_gather(table_hbm, idx_hbm, out_hbm, idx_v, rows_v, sem):
    wid = lax.axis_index("s") * nc + lax.axis_index("c")

    def body(i, carry):
      base = wid * per_w + i * SC_CHUNK
      pltpu.sync_copy(idx_hbm.at[pl.ds(base, SC_CHUNK)], idx_v)
      pltpu.async_copy(table_hbm.at[idx_v], rows_v, sem).wait()
      pltpu.sync_copy(rows_v, out_hbm.at[pl.ds(base, SC_CHUNK)])
      return carry

    lax.fori_loop(0, iters, body, 0)

  return sc_gather


def _leaky(x):
  return jnp.where(x >= 0, x, 0.2 * x)


def _mp_body(*refs, n_pos):
  pos_refs = refs[:n_pos]
  x_ref, wa_ref, w_ref, mpw_ref, mpb_ref, h_ref, sem_ref = refs[n_pos:]
  i = pl.program_id(0)

  pooled = pos_refs[0][...]
  for r in pos_refs[1:]:
    pooled = jnp.maximum(pooled, r[...])

  wa = wa_ref[...]
  s_p = jnp.sum(pooled * wa[:, None, :], axis=-1)
  s_h = jnp.sum(x_ref[...] * wa, axis=-1, keepdims=True)
  e = _leaky(s_p + s_h)
  m = jnp.max(e, axis=-1, keepdims=True)
  ex = jnp.exp(e - m)
  alpha = ex / jnp.sum(ex, axis=-1, keepdims=True)

  ctx = jnp.sum(pooled * alpha[:, :, None], axis=1)
  h = lax.dot_general(ctx, w_ref[...], (((1,), (1,)), ((), ())),
                      preferred_element_type=jnp.float32)
  h_ref[...] = h

  t = jnp.tanh(
      lax.dot_general(h, mpw_ref[...], (((1,), (1,)), ((), ())),
                      preferred_element_type=jnp.float32) + mpb_ref[...])
  ts = jnp.sum(t, axis=0, keepdims=True)

  @pl.when(i == 0)
  def _():
    sem_ref[...] = jnp.zeros_like(sem_ref)

  sem_ref[...] += jnp.broadcast_to(ts, sem_ref.shape)


def _run_metapath(g3, offs, x_anchor, wvec, w, mpw, mpb, n, blk):
  n_pos = len(offs)
  nblk = n // blk
  in_specs = []
  for off in offs:
    in_specs.append(
        pl.BlockSpec((blk, g3.shape[1], DIM),
                     lambda i, o=off: (o * nblk + i, 0, 0)))
  in_specs += [
      pl.BlockSpec((blk, DIM), lambda i: (i, 0)),
      pl.BlockSpec((1, DIM), lambda i: (0, 0)),
      pl.BlockSpec((DIM, DIM), lambda i: (0, 0)),
      pl.BlockSpec((DIM, DIM), lambda i: (0, 0)),
      pl.BlockSpec((1, DIM), lambda i: (0, 0)),
  ]
  h, sem = pl.pallas_call(
      functools.partial(_mp_body, n_pos=n_pos),
      grid=(nblk,),
      in_specs=in_specs,
      out_specs=[
          pl.BlockSpec((blk, DIM), lambda i: (i, 0)),
          pl.BlockSpec((8, DIM), lambda i: (0, 0)),
      ],
      out_shape=[
          jax.ShapeDtypeStruct((n, DIM), jnp.float32),
          jax.ShapeDtypeStruct((8, DIM), jnp.float32),
      ],
  )(g3, *([g3] * (n_pos - 1)), x_anchor, wvec, w, mpw, mpb)
  return h, sem[0]


def _combine_body(h0, h1, h2, h3, h4, h5, bvec, od, oi, og):
  b = bvec[...]
  od[...] = h0[...] * b[0:1, :] + h1[...] * b[1:2, :]
  oi[...] = h2[...] * b[2:3, :] + h3[...] * b[3:4, :] + h4[...] * b[4:5, :]
  og[...] = h5[...] * b[5:6, :]


def kernel(X_drug, X_dis, X_gene, insts_rdr, insts_rdgdr, insts_dgd,
           insts_drd, insts_drrd, insts_gdrdg, W_rdr, a_rdr, W_rdgdr,
           a_rdgdr, W_dgd, a_dgd, W_drd, a_drd, W_drrd, a_drrd, W_gdrdg,
           a_gdrdg, mpW_drug, mpb_drug, mpq_drug, mpW_dis, mpb_dis,
           mpq_dis, mpW_gene, mpb_gene, mpq_gene):
  n = X_drug.shape[0]
  k = insts_rdr.shape[1]
  xs = [X_drug, X_dis, X_gene]

  paths = [
      (insts_rdr, (0, 1, 0), W_rdr, a_rdr, 0, mpW_drug, mpb_drug),
      (insts_rdgdr, (0, 1, 2, 1, 0), W_rdgdr, a_rdgdr, 0, mpW_drug, mpb_drug),
      (insts_dgd, (1, 2, 1), W_dgd, a_dgd, 1, mpW_dis, mpb_dis),
      (insts_drd, (1, 0, 1), W_drd, a_drd, 1, mpW_dis, mpb_dis),
      (insts_drrd, (1, 0, 0, 1), W_drrd, a_drrd, 1, mpW_dis, mpb_dis),
      (insts_gdrdg, (2, 1, 0, 1, 2), W_gdrdg, a_gdrdg, 2, mpW_gene, mpb_gene),
  ]

  x_all = jnp.concatenate(xs, axis=0)

  idx_parts = []
  offs_per_path = []
  pos = 0
  for insts, tids, *_ in paths:
    offs = []
    for j, tid in enumerate(tids):
      idx_parts.append(
          (insts[:, :, j].astype(jnp.int32) + tid * n).reshape(-1))
      offs.append(pos)
      pos += 1
    offs_per_path.append(offs)
  total = pos * n * k

  n_workers = 32
  unit = n_workers * SC_CHUNK
  total_pad = total
  while total_pad % unit or total_pad % k:
    total_pad += unit - (total_pad % unit)
    if total_pad % k:
      total_pad += k - (total_pad % k)
  idx_all = jnp.concatenate(
      idx_parts + [jnp.zeros((total_pad - total,), jnp.int32)])

  gathered = _build_sc_gather(total_pad, n_workers,
                              x_all.shape[0])(x_all, idx_all)
  g3 = gathered.reshape(total_pad // k, k, DIM)

  blk = 80
  hs, sems = [], []
  for (insts, tids, w, a, anchor, mpw, mpb), offs in zip(paths,
                                                         offs_per_path):
    wvec = (a[0] @ w[0]).reshape(1, DIM)
    h, sem = _run_metapath(g3, offs, xs[anchor], wvec, w[0], mpw,
                           mpb.reshape(1, DIM), n, blk)
    hs.append(h)
    sems.append(sem / n)

  def betas(sem_list, q):
    sc = jnp.stack([jnp.sum(jnp.tanh(s) * q) for s in sem_list])
    return jax.nn.softmax(sc, axis=0)

  beta_drug = betas(sems[0:2], mpq_drug)
  beta_dis = betas(sems[2:5], mpq_dis)
  beta_gene = betas(sems[5:6], mpq_gene)

  bvec = jnp.broadcast_to(
      jnp.concatenate([beta_drug, beta_dis, beta_gene,
                       jnp.zeros((2,), jnp.float32)])[:, None],
      (8, DIM))

  nblk = n // blk
  h_specs = [pl.BlockSpec((blk, DIM), lambda i: (i, 0)) for _ in range(6)]
  out_spec = pl.BlockSpec((blk, DIM), lambda i: (i, 0))
  h_drug, h_dis, h_gene = pl.pallas_call(
      _combine_body,
      grid=(nblk,),
      in_specs=h_specs + [pl.BlockSpec((8, DIM), lambda i: (0, 0))],
      out_specs=[out_spec, out_spec, out_spec],
      out_shape=[jax.ShapeDtypeStruct((n, DIM), jnp.float32)] * 3,
  )(*hs, bvec)

  return (h_drug, h_dis, h_gene, beta_drug, beta_dis, beta_gene)


# speedup vs baseline: 1.9836x; 1.2132x over previous
import functools

import jax
import jax.numpy as jnp
from jax import lax
from jax.experimental import pallas as pl
from jax.experimental.pallas import tpu as pltpu
from jax.experimental.pallas import tpu_sc as plsc

DIM = 128
SC_CHUNK = 128


SC_GROUP = 5


def _build_sc_gather(total_pad, n_workers, table_rows):
  per_w = total_pad // n_workers
  group_rows = SC_GROUP * SC_CHUNK
  groups = per_w // group_rows
  mesh = plsc.VectorSubcoreMesh(core_axis_name="c", subcore_axis_name="s")
  info = plsc.get_sparse_core_info()
  nc = info.num_cores

  @functools.partial(
      pl.kernel,
      mesh=mesh,
      out_type=jax.ShapeDtypeStruct((total_pad, DIM), jnp.float32),
      scratch_types=[
          pltpu.VMEM((group_rows,), jnp.int32),
      ] + [pltpu.VMEM((SC_CHUNK, DIM), jnp.float32)] * SC_GROUP + [
          pltpu.SemaphoreType.DMA,
      ],
  )
  def sc_gather(table_hbm, idx_hbm, out_hbm, idx_v, *rest):
    rows = rest[:SC_GROUP]
    sem = rest[SC_GROUP]
    wid = lax.axis_index("s") * nc + lax.axis_index("c")

    def body(g, carry):
      base = wid * per_w + g * group_rows
      pltpu.sync_copy(idx_hbm.at[pl.ds(base, group_rows)], idx_v)
      handles = []
      for b in range(SC_GROUP):
        handles.append(
            pltpu.async_copy(
                table_hbm.at[idx_v.at[pl.ds(b * SC_CHUNK, SC_CHUNK)]],
                rows[b], sem))
      for b in range(SC_GROUP):
        handles[b].wait()
        pltpu.sync_copy(rows[b],
                        out_hbm.at[pl.ds(base + b * SC_CHUNK, SC_CHUNK)])
      return carry

    lax.fori_loop(0, groups, body, 0)

  return sc_gather


def _leaky(x):
  return jnp.where(x >= 0, x, 0.2 * x)


def _mp_body(*refs, n_pos):
  pos_refs = refs[:n_pos]
  x_ref, wa_ref, w_ref, mpw_ref, mpb_ref, h_ref, sem_ref = refs[n_pos:]
  i = pl.program_id(0)

  pooled = pos_refs[0][...]
  for r in pos_refs[1:]:
    pooled = jnp.maximum(pooled, r[...])

  wa = wa_ref[...]
  s_p = jnp.sum(pooled * wa[:, None, :], axis=-1)
  s_h = jnp.sum(x_ref[...] * wa, axis=-1, keepdims=True)
  e = _leaky(s_p + s_h)
  m = jnp.max(e, axis=-1, keepdims=True)
  ex = jnp.exp(e - m)
  alpha = ex / jnp.sum(ex, axis=-1, keepdims=True)

  ctx = jnp.sum(pooled * alpha[:, :, None], axis=1)
  h = lax.dot_general(ctx, w_ref[...], (((1,), (1,)), ((), ())),
                      preferred_element_type=jnp.float32)
  h_ref[...] = h

  t = jnp.tanh(
      lax.dot_general(h, mpw_ref[...], (((1,), (1,)), ((), ())),
                      preferred_element_type=jnp.float32) + mpb_ref[...])
  ts = jnp.sum(t, axis=0, keepdims=True)

  @pl.when(i == 0)
  def _():
    sem_ref[...] = jnp.zeros_like(sem_ref)

  sem_ref[...] += jnp.broadcast_to(ts, sem_ref.shape)


def _run_metapath(g3, offs, x_anchor, wvec, w, mpw, mpb, n, blk):
  n_pos = len(offs)
  nblk = n // blk
  in_specs = []
  for off in offs:
    in_specs.append(
        pl.BlockSpec((blk, g3.shape[1], DIM),
                     lambda i, o=off: (o * nblk + i, 0, 0)))
  in_specs += [
      pl.BlockSpec((blk, DIM), lambda i: (i, 0)),
      pl.BlockSpec((1, DIM), lambda i: (0, 0)),
      pl.BlockSpec((DIM, DIM), lambda i: (0, 0)),
      pl.BlockSpec((DIM, DIM), lambda i: (0, 0)),
      pl.BlockSpec((1, DIM), lambda i: (0, 0)),
  ]
  h, sem = pl.pallas_call(
      functools.partial(_mp_body, n_pos=n_pos),
      grid=(nblk,),
      in_specs=in_specs,
      out_specs=[
          pl.BlockSpec((blk, DIM), lambda i: (i, 0)),
          pl.BlockSpec((8, DIM), lambda i: (0, 0)),
      ],
      out_shape=[
          jax.ShapeDtypeStruct((n, DIM), jnp.float32),
          jax.ShapeDtypeStruct((8, DIM), jnp.float32),
      ],
  )(g3, *([g3] * (n_pos - 1)), x_anchor, wvec, w, mpw, mpb)
  return h, sem[0]


def _combine_body(h0, h1, h2, h3, h4, h5, bvec, od, oi, og):
  b = bvec[...]
  od[...] = h0[...] * b[0:1, :] + h1[...] * b[1:2, :]
  oi[...] = h2[...] * b[2:3, :] + h3[...] * b[3:4, :] + h4[...] * b[4:5, :]
  og[...] = h5[...] * b[5:6, :]


def kernel(X_drug, X_dis, X_gene, insts_rdr, insts_rdgdr, insts_dgd,
           insts_drd, insts_drrd, insts_gdrdg, W_rdr, a_rdr, W_rdgdr,
           a_rdgdr, W_dgd, a_dgd, W_drd, a_drd, W_drrd, a_drrd, W_gdrdg,
           a_gdrdg, mpW_drug, mpb_drug, mpq_drug, mpW_dis, mpb_dis,
           mpq_dis, mpW_gene, mpb_gene, mpq_gene):
  n = X_drug.shape[0]
  k = insts_rdr.shape[1]
  xs = [X_drug, X_dis, X_gene]

  paths = [
      (insts_rdr, (0, 1, 0), W_rdr, a_rdr, 0, mpW_drug, mpb_drug),
      (insts_rdgdr, (0, 1, 2, 1, 0), W_rdgdr, a_rdgdr, 0, mpW_drug, mpb_drug),
      (insts_dgd, (1, 2, 1), W_dgd, a_dgd, 1, mpW_dis, mpb_dis),
      (insts_drd, (1, 0, 1), W_drd, a_drd, 1, mpW_dis, mpb_dis),
      (insts_drrd, (1, 0, 0, 1), W_drrd, a_drrd, 1, mpW_dis, mpb_dis),
      (insts_gdrdg, (2, 1, 0, 1, 2), W_gdrdg, a_gdrdg, 2, mpW_gene, mpb_gene),
  ]

  x_all = jnp.concatenate(xs, axis=0)

  idx_parts = []
  offs_per_path = []
  pos = 0
  for insts, tids, *_ in paths:
    offs = []
    for j, tid in enumerate(tids):
      idx_parts.append(
          (insts[:, :, j].astype(jnp.int32) + tid * n).reshape(-1))
      offs.append(pos)
      pos += 1
    offs_per_path.append(offs)
  total = pos * n * k

  n_workers = 32
  unit = n_workers * SC_CHUNK * SC_GROUP
  total_pad = total
  while total_pad % unit or total_pad % k:
    total_pad += unit - (total_pad % unit)
    if total_pad % k:
      total_pad += k - (total_pad % k)
  idx_all = jnp.concatenate(
      idx_parts + [jnp.zeros((total_pad - total,), jnp.int32)])

  gathered = _build_sc_gather(total_pad, n_workers,
                              x_all.shape[0])(x_all, idx_all)
  g3 = gathered.reshape(total_pad // k, k, DIM)

  blk = 80
  hs, sems = [], []
  for (insts, tids, w, a, anchor, mpw, mpb), offs in zip(paths,
                                                         offs_per_path):
    wvec = (a[0] @ w[0]).reshape(1, DIM)
    h, sem = _run_metapath(g3, offs, xs[anchor], wvec, w[0], mpw,
                           mpb.reshape(1, DIM), n, blk)
    hs.append(h)
    sems.append(sem / n)

  def betas(sem_list, q):
    sc = jnp.stack([jnp.sum(jnp.tanh(s) * q) for s in sem_list])
    return jax.nn.softmax(sc, axis=0)

  beta_drug = betas(sems[0:2], mpq_drug)
  beta_dis = betas(sems[2:5], mpq_dis)
  beta_gene = betas(sems[5:6], mpq_gene)

  bvec = jnp.broadcast_to(
      jnp.concatenate([beta_drug, beta_dis, beta_gene,
                       jnp.zeros((2,), jnp.float32)])[:, None],
      (8, DIM))

  nblk = n // blk
  h_specs = [pl.BlockSpec((blk, DIM), lambda i: (i, 0)) for _ in range(6)]
  out_spec = pl.BlockSpec((blk, DIM), lambda i: (i, 0))
  h_drug, h_dis, h_gene = pl.pallas_call(
      _combine_body,
      grid=(nblk,),
      in_specs=h_specs + [pl.BlockSpec((8, DIM), lambda i: (0, 0))],
      out_specs=[out_spec, out_spec, out_spec],
      out_shape=[jax.ShapeDtypeStruct((n, DIM), jnp.float32)] * 3,
  )(*hs, bvec)

  return (h_drug, h_dis, h_gene, beta_drug, beta_dis, beta_gene)


# speedup vs baseline: 1.9931x; 1.0048x over previous
import functools

import jax
import jax.numpy as jnp
from jax import lax
from jax.experimental import pallas as pl
from jax.experimental.pallas import tpu as pltpu
from jax.experimental.pallas import tpu_sc as plsc

DIM = 128
SC_CHUNK = 128


SC_GROUP = 5


def _build_sc_gather(total_pad, n_workers, table_rows):
  per_w = total_pad // n_workers
  group_rows = SC_GROUP * SC_CHUNK
  groups = per_w // group_rows
  mesh = plsc.VectorSubcoreMesh(core_axis_name="c", subcore_axis_name="s")
  info = plsc.get_sparse_core_info()
  nc = info.num_cores

  @functools.partial(
      pl.kernel,
      mesh=mesh,
      out_type=jax.ShapeDtypeStruct((total_pad, DIM), jnp.float32),
      scratch_types=[
          pltpu.VMEM((group_rows,), jnp.int32),
      ] + [pltpu.VMEM((SC_CHUNK, DIM), jnp.float32)] * SC_GROUP + [
          pltpu.SemaphoreType.DMA,
          pltpu.SemaphoreType.DMA,
      ],
  )
  def sc_gather(table_hbm, idx_hbm, out_hbm, idx_v, *rest):
    rows = rest[:SC_GROUP]
    gsem, wsem = rest[SC_GROUP], rest[SC_GROUP + 1]
    wid = lax.axis_index("s") * nc + lax.axis_index("c")

    def drain_writebacks(group_base):
      for b in range(SC_GROUP):
        pltpu.make_async_copy(
            rows[b],
            out_hbm.at[pl.ds(group_base + b * SC_CHUNK, SC_CHUNK)],
            wsem).wait()

    def body(g, carry):
      base = wid * per_w + g * group_rows
      pltpu.sync_copy(idx_hbm.at[pl.ds(base, group_rows)], idx_v)

      @pl.when(g > 0)
      def _():
        drain_writebacks(base - group_rows)

      handles = []
      for b in range(SC_GROUP):
        handles.append(
            pltpu.async_copy(
                table_hbm.at[idx_v.at[pl.ds(b * SC_CHUNK, SC_CHUNK)]],
                rows[b], gsem))
      for b in range(SC_GROUP):
        handles[b].wait()
        pltpu.async_copy(rows[b],
                         out_hbm.at[pl.ds(base + b * SC_CHUNK, SC_CHUNK)],
                         wsem)
      return carry

    lax.fori_loop(0, groups, body, 0)
    drain_writebacks(wid * per_w + (groups - 1) * group_rows)

  return sc_gather


def _leaky(x):
  return jnp.where(x >= 0, x, 0.2 * x)


def _mp_body(*refs, n_pos):
  pos_refs = refs[:n_pos]
  x_ref, wa_ref, w_ref, mpw_ref, mpb_ref, h_ref, sem_ref = refs[n_pos:]
  i = pl.program_id(0)

  pooled = pos_refs[0][...]
  for r in pos_refs[1:]:
    pooled = jnp.maximum(pooled, r[...])

  wa = wa_ref[...]
  s_p = jnp.sum(pooled * wa[:, None, :], axis=-1)
  s_h = jnp.sum(x_ref[...] * wa, axis=-1, keepdims=True)
  e = _leaky(s_p + s_h)
  m = jnp.max(e, axis=-1, keepdims=True)
  ex = jnp.exp(e - m)
  alpha = ex / jnp.sum(ex, axis=-1, keepdims=True)

  ctx = jnp.sum(pooled * alpha[:, :, None], axis=1)
  h = lax.dot_general(ctx, w_ref[...], (((1,), (1,)), ((), ())),
                      preferred_element_type=jnp.float32)
  h_ref[...] = h

  t = jnp.tanh(
      lax.dot_general(h, mpw_ref[...], (((1,), (1,)), ((), ())),
                      preferred_element_type=jnp.float32) + mpb_ref[...])
  ts = jnp.sum(t, axis=0, keepdims=True)

  @pl.when(i == 0)
  def _():
    sem_ref[...] = jnp.zeros_like(sem_ref)

  sem_ref[...] += jnp.broadcast_to(ts, sem_ref.shape)


def _run_metapath(g3, offs, x_anchor, wvec, w, mpw, mpb, n, blk):
  n_pos = len(offs)
  nblk = n // blk
  in_specs = []
  for off in offs:
    in_specs.append(
        pl.BlockSpec((blk, g3.shape[1], DIM),
                     lambda i, o=off: (o * nblk + i, 0, 0)))
  in_specs += [
      pl.BlockSpec((blk, DIM), lambda i: (i, 0)),
      pl.BlockSpec((1, DIM), lambda i: (0, 0)),
      pl.BlockSpec((DIM, DIM), lambda i: (0, 0)),
      pl.BlockSpec((DIM, DIM), lambda i: (0, 0)),
      pl.BlockSpec((1, DIM), lambda i: (0, 0)),
  ]
  h, sem = pl.pallas_call(
      functools.partial(_mp_body, n_pos=n_pos),
      grid=(nblk,),
      in_specs=in_specs,
      out_specs=[
          pl.BlockSpec((blk, DIM), lambda i: (i, 0)),
          pl.BlockSpec((8, DIM), lambda i: (0, 0)),
      ],
      out_shape=[
          jax.ShapeDtypeStruct((n, DIM), jnp.float32),
          jax.ShapeDtypeStruct((8, DIM), jnp.float32),
      ],
  )(g3, *([g3] * (n_pos - 1)), x_anchor, wvec, w, mpw, mpb)
  return h, sem[0]


def _combine_body(h0, h1, h2, h3, h4, h5, bvec, od, oi, og):
  b = bvec[...]
  od[...] = h0[...] * b[0:1, :] + h1[...] * b[1:2, :]
  oi[...] = h2[...] * b[2:3, :] + h3[...] * b[3:4, :] + h4[...] * b[4:5, :]
  og[...] = h5[...] * b[5:6, :]


def kernel(X_drug, X_dis, X_gene, insts_rdr, insts_rdgdr, insts_dgd,
           insts_drd, insts_drrd, insts_gdrdg, W_rdr, a_rdr, W_rdgdr,
           a_rdgdr, W_dgd, a_dgd, W_drd, a_drd, W_drrd, a_drrd, W_gdrdg,
           a_gdrdg, mpW_drug, mpb_drug, mpq_drug, mpW_dis, mpb_dis,
           mpq_dis, mpW_gene, mpb_gene, mpq_gene):
  n = X_drug.shape[0]
  k = insts_rdr.shape[1]
  xs = [X_drug, X_dis, X_gene]

  paths = [
      (insts_rdr, (0, 1, 0), W_rdr, a_rdr, 0, mpW_drug, mpb_drug),
      (insts_rdgdr, (0, 1, 2, 1, 0), W_rdgdr, a_rdgdr, 0, mpW_drug, mpb_drug),
      (insts_dgd, (1, 2, 1), W_dgd, a_dgd, 1, mpW_dis, mpb_dis),
      (insts_drd, (1, 0, 1), W_drd, a_drd, 1, mpW_dis, mpb_dis),
      (insts_drrd, (1, 0, 0, 1), W_drrd, a_drrd, 1, mpW_dis, mpb_dis),
      (insts_gdrdg, (2, 1, 0, 1, 2), W_gdrdg, a_gdrdg, 2, mpW_gene, mpb_gene),
  ]

  x_all = jnp.concatenate(xs, axis=0)

  idx_parts = []
  offs_per_path = []
  pos = 0
  for insts, tids, *_ in paths:
    offs = []
    for j, tid in enumerate(tids):
      idx_parts.append(
          (insts[:, :, j].astype(jnp.int32) + tid * n).reshape(-1))
      offs.append(pos)
      pos += 1
    offs_per_path.append(offs)
  total = pos * n * k

  n_workers = 32
  unit = n_workers * SC_CHUNK * SC_GROUP
  total_pad = total
  while total_pad % unit or total_pad % k:
    total_pad += unit - (total_pad % unit)
    if total_pad % k:
      total_pad += k - (total_pad % k)
  idx_all = jnp.concatenate(
      idx_parts + [jnp.zeros((total_pad - total,), jnp.int32)])

  gathered = _build_sc_gather(total_pad, n_workers,
                              x_all.shape[0])(x_all, idx_all)
  g3 = gathered.reshape(total_pad // k, k, DIM)

  blk = 80
  hs, sems = [], []
  for (insts, tids, w, a, anchor, mpw, mpb), offs in zip(paths,
                                                         offs_per_path):
    wvec = (a[0] @ w[0]).reshape(1, DIM)
    h, sem = _run_metapath(g3, offs, xs[anchor], wvec, w[0], mpw,
                           mpb.reshape(1, DIM), n, blk)
    hs.append(h)
    sems.append(sem / n)

  def betas(sem_list, q):
    sc = jnp.stack([jnp.sum(jnp.tanh(s) * q) for s in sem_list])
    return jax.nn.softmax(sc, axis=0)

  beta_drug = betas(sems[0:2], mpq_drug)
  beta_dis = betas(sems[2:5], mpq_dis)
  beta_gene = betas(sems[5:6], mpq_gene)

  bvec = jnp.broadcast_to(
      jnp.concatenate([beta_drug, beta_dis, beta_gene,
                       jnp.zeros((2,), jnp.float32)])[:, None],
      (8, DIM))

  nblk = n // blk
  h_specs = [pl.BlockSpec((blk, DIM), lambda i: (i, 0)) for _ in range(6)]
  out_spec = pl.BlockSpec((blk, DIM), lambda i: (i, 0))
  h_drug, h_dis, h_gene = pl.pallas_call(
      _combine_body,
      grid=(nblk,),
      in_specs=h_specs + [pl.BlockSpec((8, DIM), lambda i: (0, 0))],
      out_specs=[out_spec, out_spec, out_spec],
      out_shape=[jax.ShapeDtypeStruct((n, DIM), jnp.float32)] * 3,
  )(*hs, bvec)

  return (h_drug, h_dis, h_gene, beta_drug, beta_dis, beta_gene)
